# Initial kernel scaffold; baseline (speedup 1.0000x reference)
import jax
import jax.numpy as jnp
from jax.experimental import pallas as pl


def kernel(X, table, bias):
    raise NotImplementedError("write your pallas kernel here")


# speedup vs baseline: 1.2918x; 1.2918x over previous
import functools

import jax
import jax.numpy as jnp
from jax import lax
from jax.experimental import pallas as pl
from jax.experimental.pallas import tpu as pltpu
from jax.experimental.pallas import tpu_sc as plsc

B = 16384
F = 26
NC = 2
NS = 16
NW = NC * NS
BPW = B // NW
IPW = BPW * F
CHUNKS = BPW // 16


def _lr_kernel(x_hbm, t_hbm, bias_hbm, out_hbm, idx_v, vals_v, acc_v, bias_v,
               sem):
    wid = lax.axis_index("s") * NC + lax.axis_index("c")
    base = wid * IPW

    pltpu.sync_copy(x_hbm.at[pl.ds(base, IPW)], idx_v)
    pltpu.sync_copy(bias_hbm, bias_v)
    pltpu.async_copy(t_hbm.at[idx_v], vals_v, sem).wait()

    lane = lax.iota(jnp.int32, 16)
    bias_vec = bias_v[...]

    def chunk_body(c, _):
        rowbase = c * (16 * F)
        idx0 = rowbase + lane * F
        acc = bias_vec
        for f in range(F):
            acc = acc + plsc.load_gather(vals_v, [idx0 + f])
        acc_v[pl.ds(c * 16, 16)] = acc
        return 0

    lax.fori_loop(0, CHUNKS, chunk_body, 0)
    pltpu.sync_copy(acc_v, out_hbm.at[pl.ds(wid * BPW, BPW)])


@jax.jit
def _lr(x_flat, t_flat, bias16):
    mesh = plsc.VectorSubcoreMesh(core_axis_name="c", subcore_axis_name="s")
    f = functools.partial(
        pl.kernel,
        out_type=jax.ShapeDtypeStruct((B,), jnp.float32),
        mesh=mesh,
        scratch_types=[
            pltpu.VMEM((IPW,), jnp.int32),
            pltpu.VMEM((IPW,), jnp.float32),
            pltpu.VMEM((BPW,), jnp.float32),
            pltpu.VMEM((16,), jnp.float32),
            pltpu.SemaphoreType.DMA,
        ],
        compiler_params=pltpu.CompilerParams(needs_layout_passes=False),
    )(_lr_kernel)
    return f(x_flat, t_flat, bias16)


def kernel(X, table, bias):
    x_flat = X.reshape(-1)
    t_flat = table.reshape(-1)
    bias16 = jnp.broadcast_to(bias, (16,))
    out = _lr(x_flat, t_flat, bias16)
    return out.reshape(B, 1)


# speedup vs baseline: 1.2933x; 1.0011x over previous
import functools

import jax
import jax.numpy as jnp
from jax import lax
from jax.experimental import pallas as pl
from jax.experimental.pallas import tpu as pltpu
from jax.experimental.pallas import tpu_sc as plsc

B = 16384
F = 26
NC = 2
NS = 16
NW = NC * NS
BPW = B // NW
IPW = BPW * F
CHUNKS = BPW // 16


NQ = 4
QI = IPW // NQ


def _lr_kernel(x_hbm, t_hbm, bias_hbm, out_hbm, idx_v, vals_v, acc_v, bias_v,
               sem):
    wid = lax.axis_index("s") * NC + lax.axis_index("c")
    base = wid * IPW

    pltpu.sync_copy(x_hbm.at[pl.ds(base, IPW)], idx_v)
    pltpu.sync_copy(bias_hbm, bias_v)
    copies = [
        pltpu.async_copy(
            t_hbm.at[idx_v.at[pl.ds(q * QI, QI)]],
            vals_v.at[pl.ds(q * QI, QI)],
            sem,
        )
        for q in range(NQ)
    ]
    for c in copies:
        c.wait()

    lane = lax.iota(jnp.int32, 16)
    bias_vec = bias_v[...]

    def chunk_body(c, _):
        rowbase = c * (16 * F)
        idx0 = rowbase + lane * F
        acc = bias_vec
        for f in range(F):
            acc = acc + plsc.load_gather(vals_v, [idx0 + f])
        acc_v[pl.ds(c * 16, 16)] = acc
        return 0

    lax.fori_loop(0, CHUNKS, chunk_body, 0)
    pltpu.sync_copy(acc_v, out_hbm.at[pl.ds(wid * BPW, BPW)])


@jax.jit
def _lr(x_flat, t_flat, bias16):
    mesh = plsc.VectorSubcoreMesh(core_axis_name="c", subcore_axis_name="s")
    f = functools.partial(
        pl.kernel,
        out_type=jax.ShapeDtypeStruct((B,), jnp.float32),
        mesh=mesh,
        scratch_types=[
            pltpu.VMEM((IPW,), jnp.int32),
            pltpu.VMEM((IPW,), jnp.float32),
            pltpu.VMEM((BPW,), jnp.float32),
            pltpu.VMEM((16,), jnp.float32),
            pltpu.SemaphoreType.DMA,
        ],
        compiler_params=pltpu.CompilerParams(needs_layout_passes=False),
    )(_lr_kernel)
    return f(x_flat, t_flat, bias16)


def kernel(X, table, bias):
    x_flat = X.reshape(-1)
    t_flat = table.reshape(-1)
    bias16 = jnp.broadcast_to(bias, (16,))
    out = _lr(x_flat, t_flat, bias16)
    return out.reshape(B, 1)
